# baseline (speedup 1.0000x reference)
import functools

import jax
import jax.numpy as jnp
from jax import lax
from jax.experimental import pallas as pl
from jax.experimental.pallas import tpu as pltpu
from jax.experimental.pallas import tpu_sc as plsc

N_NODES = 10000
F_IN = 128
HID = 64
G_PER = 16
NCAT = 2 * N_NODES
NPAD = 20096
E_EDGES = 320000
ECAT = 2 * E_EDGES
EPAD = 655360
NW = 32
EPW = EPAD // NW
K = 128
CHUNKS = EPW // K
HC = CHUNKS // 2
SEG = 32
CH_A = 320
CH_B = EPAD // (16 * K) - CH_A
RPS = NPAD // 16

_sc_mesh = plsc.VectorSubcoreMesh(core_axis_name="c", subcore_axis_name="s")


@functools.partial(
    pl.kernel,
    out_type=jax.ShapeDtypeStruct((NW, NPAD), jnp.float32),
    mesh=_sc_mesh,
    compiler_params=pltpu.CompilerParams(needs_layout_passes=False),
    scratch_types=[
        pltpu.VMEM((NPAD,), jnp.float32),
        pltpu.VMEM((CHUNKS, K), jnp.int32),
    ],
)
def _sc_degree(dst_hbm, out_hbm, deg_v, idx_v):
    c = lax.axis_index("c")
    s = lax.axis_index("s")
    wid = s * 2 + c
    pltpu.sync_copy(dst_hbm.at[wid], idx_v)

    def zero_body(i, carry):
        deg_v[pl.ds(i * 16, 16)] = jnp.zeros((16,), jnp.float32)
        return carry

    lax.fori_loop(0, NPAD // 16, zero_body, 0)

    ones16 = jnp.ones((16,), jnp.float32)

    def chunk_body(j, carry):
        for k in range(K // 16):
            iv = idx_v[j, pl.ds(k * 16, 16)]
            plsc.addupdate_scatter(deg_v, [iv], ones16)
        return carry

    lax.fori_loop(0, CHUNKS, chunk_body, 0)
    pltpu.sync_copy(deg_v, out_hbm.at[wid])


@functools.partial(
    pl.kernel,
    out_type=jax.ShapeDtypeStruct((1, NPAD, HID), jnp.float32),
    mesh=_sc_mesh,
    compiler_params=pltpu.CompilerParams(
        needs_layout_passes=False, use_tc_tiling_on_sc=False),
    scratch_types=[
        pltpu.VMEM_SHARED((NPAD, HID), jnp.float32),
        pltpu.VMEM((K, HID), jnp.float32),
        pltpu.VMEM((K, HID), jnp.float32),
        pltpu.VMEM((SEG, K), jnp.int32),
        pltpu.VMEM((SEG, K), jnp.int32),
        pltpu.SemaphoreType.DMA,
        pltpu.SemaphoreType.DMA,
    ],
)
def _sc_scatter(m_hbm, srca_hbm, dsta_hbm,
                out_hbm, acc, buf0, buf1, sidx, didx, sem0, sem1):
    c = lax.axis_index("c")
    s = lax.axis_index("s")
    r0 = s * RPS

    def zrow_body(i, carry):
        for f in range(HID // 16):
            buf0[i, pl.ds(f * 16, 16)] = jnp.zeros((16,), jnp.float32)
        return carry

    lax.fori_loop(0, K, zrow_body, 0)
    for z in range(RPS // K):
        pltpu.sync_copy(buf0, acc.at[pl.ds(r0 + z * K, K)])
    rem = RPS - (RPS // K) * K
    if rem:
        pltpu.sync_copy(buf0.at[pl.ds(0, rem)],
                        acc.at[pl.ds(r0 + (RPS // K) * K, rem)])
    plsc.subcore_barrier()

    def start(ch, buf, sem):
        pltpu.async_copy(m_hbm.at[sidx.at[ch]], buf, sem)

    def drain(buf, sem):
        pltpu.make_async_copy(m_hbm.at[pl.ds(0, K)], buf, sem).wait()

    def scat(ch, buf):
        pltpu.sync_copy(buf, acc.at[didx.at[ch]], add=True)

    def run(src_hbm, dst_hbm, nch):
        for g in range(nch // SEG):
            pltpu.sync_copy(src_hbm.at[s, pl.ds(g * SEG, SEG)], sidx)
            pltpu.sync_copy(dst_hbm.at[s, pl.ds(g * SEG, SEG)], didx)
            start(0, buf0, sem0)

            def body(i, carry):
                start(2 * i + 1, buf1, sem1)
                drain(buf0, sem0)
                scat(2 * i, buf0)
                start(2 * i + 2, buf0, sem0)
                drain(buf1, sem1)
                scat(2 * i + 1, buf1)
                return carry

            lax.fori_loop(0, SEG // 2 - 1, body, 0)
            start(SEG - 1, buf1, sem1)
            drain(buf0, sem0)
            scat(SEG - 2, buf0)
            drain(buf1, sem1)
            scat(SEG - 1, buf1)

    @pl.when(c == 0)
    def _run_a():
        run(srca_hbm, dsta_hbm, CH_A)

    plsc.subcore_barrier()

    @pl.when(c == 0)
    def _copy_out():
        nfull = RPS // K
        for z in range(nfull):
            buf = buf0 if z % 2 == 0 else buf1
            pltpu.sync_copy(acc.at[pl.ds(r0 + z * K, K)], buf)
            pltpu.sync_copy(buf, out_hbm.at[0, pl.ds(r0 + z * K, K)])
        rem = RPS - nfull * K
        if rem:
            pltpu.sync_copy(acc.at[pl.ds(r0 + nfull * K, rem)],
                            buf0.at[pl.ds(0, rem)])
            pltpu.sync_copy(buf0.at[pl.ds(0, rem)],
                            out_hbm.at[0, pl.ds(r0 + nfull * K, rem)])


def _tc_prep(xp, degp, W1):
    def body(x_ref, dp_ref, w_ref, m1_ref, dinv_ref):
        deg = jnp.sum(dp_ref[...], axis=0, keepdims=True) + 1.0
        dinv = lax.rsqrt(deg)
        dinv_ref[...] = dinv
        dcol = jnp.reshape(dinv, (NPAD, 1))
        h0 = jnp.dot(x_ref[...], w_ref[...], preferred_element_type=jnp.float32)
        m1_ref[...] = h0 * dcol

    return pl.pallas_call(
        body,
        out_shape=[
            jax.ShapeDtypeStruct((NPAD, HID), jnp.float32),
            jax.ShapeDtypeStruct((1, NPAD), jnp.float32),
        ],
    )(xp, degp, W1)


def _tc_mid(s1p, m1, dinv, W2, b1):
    def body(s_ref, m_ref, d_ref, w_ref, b_ref, m2_ref):
        dcol = jnp.reshape(d_ref[...], (NPAD, 1))
        s = s_ref[0]
        h1 = jnp.maximum(dcol * (s + m_ref[...]) + b_ref[...], 0.0)
        m2 = jnp.dot(h1, w_ref[...], preferred_element_type=jnp.float32) * dcol
        rows = lax.broadcasted_iota(jnp.int32, (NPAD, 1), 0)
        m2_ref[...] = jnp.where(rows < NCAT, m2, 0.0)

    return pl.pallas_call(
        body,
        out_shape=jax.ShapeDtypeStruct((NPAD, HID), jnp.float32),
    )(s1p, m1, dinv, W2, b1)


def _tc_final(s2p, m2, dinv, b2, batchf, fc1_W, fc1_b, fc2_W, fc2_b):
    def body(s_ref, m_ref, d_ref, b2_ref, bat_ref, f1w_ref, f1b_ref,
             f2w_ref, f2b_ref, o_ref):
        dcol = jnp.reshape(d_ref[...], (NPAD, 1))
        s = s_ref[0]
        h2 = jnp.maximum(dcol * (s + m_ref[...]) + b2_ref[...], 0.0)
        gids = lax.broadcasted_iota(jnp.int32, (2 * G_PER, 1), 0).astype(jnp.float32)
        P = (gids == bat_ref[...]).astype(jnp.float32)
        sums = jnp.dot(P, h2, preferred_element_type=jnp.float32)
        cnt = jnp.sum(P, axis=1, keepdims=True)
        pooled = sums / jnp.maximum(cnt, 1.0)
        e1 = pooled[0:G_PER]
        e2 = pooled[G_PER:2 * G_PER]
        f1w = f1w_ref[...]
        z = (jnp.dot(e1, f1w[0:HID], preferred_element_type=jnp.float32)
             + jnp.dot(e2, f1w[HID:2 * HID], preferred_element_type=jnp.float32)
             + jnp.dot(jnp.abs(e1 - e2), f1w[2 * HID:3 * HID],
                       preferred_element_type=jnp.float32)
             + f1b_ref[...])
        z = jnp.maximum(z, 0.0)
        o = jnp.dot(z, f2w_ref[...], preferred_element_type=jnp.float32) + f2b_ref[...]
        o_ref[...] = jax.nn.sigmoid(o)

    return pl.pallas_call(
        body,
        out_shape=jax.ShapeDtypeStruct((G_PER, 1), jnp.float32),
    )(s2p, m2, dinv, b2, batchf, fc1_W, fc1_b, fc2_W, fc2_b)


def kernel(x1, edge_index1, batch1, x2, edge_index2, batch2,
           W1, b1, W2, b2, fc1_W, fc1_b, fc2_W, fc2_b):
    f32 = jnp.float32
    i32 = jnp.int32
    pad_e = EPAD - ECAT
    pad_n = NPAD - NCAT
    src = jnp.concatenate([edge_index1[0], edge_index2[0] + N_NODES,
                           jnp.full((pad_e,), NCAT, i32)])
    dst = jnp.concatenate([edge_index1[1], edge_index2[1] + N_NODES,
                           jnp.full((pad_e,), NCAT, i32)])
    src_a = src.reshape(16, CH_A, K)
    dst_a = dst.reshape(16, CH_A, K)
    xp = jnp.concatenate([x1, x2, jnp.zeros((pad_n, F_IN), f32)], axis=0)
    batchf = jnp.concatenate(
        [batch1, batch2 + G_PER, jnp.full((pad_n,), 2 * G_PER + 7, i32)]
    ).astype(f32).reshape(1, NPAD)

    degp = _sc_degree(dst.reshape(NW, CHUNKS, K))
    m1, dinv = _tc_prep(xp, degp, W1)
    s1 = _sc_scatter(m1, src_a, dst_a)
    m2 = _tc_mid(s1, m1, dinv, W2, b1.reshape(1, HID))
    s2 = _sc_scatter(m2, src_a, dst_a)
    o = _tc_final(s2, m2, dinv, b2.reshape(1, HID), batchf,
                  fc1_W, fc1_b.reshape(1, HID), fc2_W, fc2_b.reshape(1, 1))
    return o[:, 0]


# speedup vs baseline: 1.3530x; 1.3530x over previous
import functools

import jax
import jax.numpy as jnp
from jax import lax
from jax.experimental import pallas as pl
from jax.experimental.pallas import tpu as pltpu
from jax.experimental.pallas import tpu_sc as plsc

N_NODES = 10000
F_IN = 128
HID = 64
G_PER = 16
NCAT = 2 * N_NODES
NPAD = 20096
E_EDGES = 320000
ECAT = 2 * E_EDGES
EPAD = 655360
NW = 32
EPW = EPAD // NW
K = 128
CHUNKS = EPW // K
HC = CHUNKS // 2
SEG = 40
CH_A = 160
CH_B = EPAD // (16 * K) - CH_A
RPS = NPAD // 16

_sc_mesh = plsc.VectorSubcoreMesh(core_axis_name="c", subcore_axis_name="s")


@functools.partial(
    pl.kernel,
    out_type=jax.ShapeDtypeStruct((NW, NPAD), jnp.float32),
    mesh=_sc_mesh,
    compiler_params=pltpu.CompilerParams(needs_layout_passes=False),
    scratch_types=[
        pltpu.VMEM((NPAD,), jnp.float32),
        pltpu.VMEM((CHUNKS, K), jnp.int32),
    ],
)
def _sc_degree(dst_hbm, out_hbm, deg_v, idx_v):
    c = lax.axis_index("c")
    s = lax.axis_index("s")
    wid = s * 2 + c
    pltpu.sync_copy(dst_hbm.at[wid], idx_v)

    def zero_body(i, carry):
        deg_v[pl.ds(i * 16, 16)] = jnp.zeros((16,), jnp.float32)
        return carry

    lax.fori_loop(0, NPAD // 16, zero_body, 0)

    ones16 = jnp.ones((16,), jnp.float32)

    def chunk_body(j, carry):
        for k in range(K // 16):
            iv = idx_v[j, pl.ds(k * 16, 16)]
            plsc.addupdate_scatter(deg_v, [iv], ones16)
        return carry

    lax.fori_loop(0, CHUNKS, chunk_body, 0)
    pltpu.sync_copy(deg_v, out_hbm.at[wid])


@functools.partial(
    pl.kernel,
    out_type=jax.ShapeDtypeStruct((2, NPAD, HID), jnp.float32),
    mesh=_sc_mesh,
    compiler_params=pltpu.CompilerParams(
        needs_layout_passes=False, use_tc_tiling_on_sc=False),
    scratch_types=[
        pltpu.VMEM_SHARED((NPAD, HID), jnp.float32),
        pltpu.VMEM((K, HID), jnp.float32),
        pltpu.VMEM((K, HID), jnp.float32),
        pltpu.VMEM((SEG, K), jnp.int32),
        pltpu.VMEM((SEG, K), jnp.int32),
        pltpu.SemaphoreType.DMA,
        pltpu.SemaphoreType.DMA,
    ],
)
def _sc_scatter(m_hbm, srca_hbm, srcb_hbm, dsta_hbm, dstb_hbm,
                out_hbm, acc, buf0, buf1, sidx, didx, sem0, sem1):
    c = lax.axis_index("c")
    s = lax.axis_index("s")
    r0 = s * RPS

    def zrow_body(i, carry):
        for f in range(HID // 16):
            buf0[i, pl.ds(f * 16, 16)] = jnp.zeros((16,), jnp.float32)
        return carry

    lax.fori_loop(0, K, zrow_body, 0)
    for z in range(RPS // K):
        pltpu.sync_copy(buf0, acc.at[pl.ds(r0 + z * K, K)])
    rem = RPS - (RPS // K) * K
    if rem:
        pltpu.sync_copy(buf0.at[pl.ds(0, rem)],
                        acc.at[pl.ds(r0 + (RPS // K) * K, rem)])
    plsc.subcore_barrier()

    def start(ch, buf, sem):
        pltpu.async_copy(m_hbm.at[sidx.at[ch]], buf, sem)

    def drain(buf, sem):
        pltpu.make_async_copy(m_hbm.at[pl.ds(0, K)], buf, sem).wait()

    def scat(ch, buf):
        pltpu.sync_copy(buf, acc.at[didx.at[ch]], add=True)

    def run(src_hbm, dst_hbm, nch):
        for g in range(nch // SEG):
            pltpu.sync_copy(src_hbm.at[s, pl.ds(g * SEG, SEG)], sidx)
            pltpu.sync_copy(dst_hbm.at[s, pl.ds(g * SEG, SEG)], didx)
            start(0, buf0, sem0)

            def body(i, carry):
                start(2 * i + 1, buf1, sem1)
                drain(buf0, sem0)
                scat(2 * i, buf0)
                start(2 * i + 2, buf0, sem0)
                drain(buf1, sem1)
                scat(2 * i + 1, buf1)
                return carry

            lax.fori_loop(0, SEG // 2 - 1, body, 0)
            start(SEG - 1, buf1, sem1)
            drain(buf0, sem0)
            scat(SEG - 2, buf0)
            drain(buf1, sem1)
            scat(SEG - 1, buf1)

    @pl.when(c == 0)
    def _run_a():
        run(srca_hbm, dsta_hbm, CH_A)

    @pl.when(c == 1)
    def _run_b():
        run(srcb_hbm, dstb_hbm, CH_B)

    plsc.subcore_barrier()
    pltpu.sync_copy(acc.at[pl.ds(r0, RPS)], out_hbm.at[c, pl.ds(r0, RPS)])


def _tc_prep(xp, degp, W1):
    def body(x_ref, dp_ref, w_ref, m1_ref, dinv_ref):
        deg = jnp.sum(dp_ref[...], axis=0, keepdims=True) + 1.0
        dinv = lax.rsqrt(deg)
        dinv_ref[...] = dinv
        dcol = jnp.reshape(dinv, (NPAD, 1))
        h0 = jnp.dot(x_ref[...], w_ref[...], preferred_element_type=jnp.float32)
        m1_ref[...] = h0 * dcol

    return pl.pallas_call(
        body,
        out_shape=[
            jax.ShapeDtypeStruct((NPAD, HID), jnp.float32),
            jax.ShapeDtypeStruct((1, NPAD), jnp.float32),
        ],
    )(xp, degp, W1)


def _tc_mid(s1p, m1, dinv, W2, b1):
    def body(s_ref, m_ref, d_ref, w_ref, b_ref, m2_ref):
        dcol = jnp.reshape(d_ref[...], (NPAD, 1))
        s = s_ref[0] + s_ref[1]
        h1 = jnp.maximum(dcol * (s + m_ref[...]) + b_ref[...], 0.0)
        m2 = jnp.dot(h1, w_ref[...], preferred_element_type=jnp.float32) * dcol
        rows = lax.broadcasted_iota(jnp.int32, (NPAD, 1), 0)
        m2_ref[...] = jnp.where(rows < NCAT, m2, 0.0)

    return pl.pallas_call(
        body,
        out_shape=jax.ShapeDtypeStruct((NPAD, HID), jnp.float32),
    )(s1p, m1, dinv, W2, b1)


def _tc_final(s2p, m2, dinv, b2, batchf, fc1_W, fc1_b, fc2_W, fc2_b):
    def body(s_ref, m_ref, d_ref, b2_ref, bat_ref, f1w_ref, f1b_ref,
             f2w_ref, f2b_ref, o_ref):
        dcol = jnp.reshape(d_ref[...], (NPAD, 1))
        s = s_ref[0] + s_ref[1]
        h2 = jnp.maximum(dcol * (s + m_ref[...]) + b2_ref[...], 0.0)
        gids = lax.broadcasted_iota(jnp.int32, (2 * G_PER, 1), 0).astype(jnp.float32)
        P = (gids == bat_ref[...]).astype(jnp.float32)
        sums = jnp.dot(P, h2, preferred_element_type=jnp.float32)
        cnt = jnp.sum(P, axis=1, keepdims=True)
        pooled = sums / jnp.maximum(cnt, 1.0)
        e1 = pooled[0:G_PER]
        e2 = pooled[G_PER:2 * G_PER]
        f1w = f1w_ref[...]
        z = (jnp.dot(e1, f1w[0:HID], preferred_element_type=jnp.float32)
             + jnp.dot(e2, f1w[HID:2 * HID], preferred_element_type=jnp.float32)
             + jnp.dot(jnp.abs(e1 - e2), f1w[2 * HID:3 * HID],
                       preferred_element_type=jnp.float32)
             + f1b_ref[...])
        z = jnp.maximum(z, 0.0)
        o = jnp.dot(z, f2w_ref[...], preferred_element_type=jnp.float32) + f2b_ref[...]
        o_ref[...] = jax.nn.sigmoid(o)

    return pl.pallas_call(
        body,
        out_shape=jax.ShapeDtypeStruct((G_PER, 1), jnp.float32),
    )(s2p, m2, dinv, b2, batchf, fc1_W, fc1_b, fc2_W, fc2_b)


def kernel(x1, edge_index1, batch1, x2, edge_index2, batch2,
           W1, b1, W2, b2, fc1_W, fc1_b, fc2_W, fc2_b):
    f32 = jnp.float32
    i32 = jnp.int32
    pad_e = EPAD - ECAT
    pad_n = NPAD - NCAT
    src = jnp.concatenate([edge_index1[0], edge_index2[0] + N_NODES,
                           jnp.full((pad_e,), NCAT, i32)])
    dst = jnp.concatenate([edge_index1[1], edge_index2[1] + N_NODES,
                           jnp.full((pad_e,), NCAT, i32)])
    ea = 16 * CH_A * K
    src_a = src[:ea].reshape(16, CH_A, K)
    src_b = src[ea:].reshape(16, CH_B, K)
    dst_a = dst[:ea].reshape(16, CH_A, K)
    dst_b = dst[ea:].reshape(16, CH_B, K)
    xp = jnp.concatenate([x1, x2, jnp.zeros((pad_n, F_IN), f32)], axis=0)
    batchf = jnp.concatenate(
        [batch1, batch2 + G_PER, jnp.full((pad_n,), 2 * G_PER + 7, i32)]
    ).astype(f32).reshape(1, NPAD)

    degp = _sc_degree(dst.reshape(NW, CHUNKS, K))
    m1, dinv = _tc_prep(xp, degp, W1)
    s1 = _sc_scatter(m1, src_a, src_b, dst_a, dst_b)
    m2 = _tc_mid(s1, m1, dinv, W2, b1.reshape(1, HID))
    s2 = _sc_scatter(m2, src_a, src_b, dst_a, dst_b)
    o = _tc_final(s2, m2, dinv, b2.reshape(1, HID), batchf,
                  fc1_W, fc1_b.reshape(1, HID), fc2_W, fc2_b.reshape(1, 1))
    return o[:, 0]


# speedup vs baseline: 1.3653x; 1.0091x over previous
import functools

import jax
import jax.numpy as jnp
from jax import lax
from jax.experimental import pallas as pl
from jax.experimental.pallas import tpu as pltpu
from jax.experimental.pallas import tpu_sc as plsc

N_NODES = 10000
F_IN = 128
HID = 64
G_PER = 16
NCAT = 2 * N_NODES
NPAD = 20096
E_EDGES = 320000
ECAT = 2 * E_EDGES
EPAD = 655360
NW = 32
EPW = EPAD // NW
K = 128
CHUNKS = EPW // K
HC = CHUNKS // 2
SEG = 80
CH_A = 160
CH_B = EPAD // (16 * K) - CH_A
RPS = NPAD // 16

_sc_mesh = plsc.VectorSubcoreMesh(core_axis_name="c", subcore_axis_name="s")


@functools.partial(
    pl.kernel,
    out_type=jax.ShapeDtypeStruct((NW, NPAD), jnp.float32),
    mesh=_sc_mesh,
    compiler_params=pltpu.CompilerParams(needs_layout_passes=False),
    scratch_types=[
        pltpu.VMEM((NPAD,), jnp.float32),
        pltpu.VMEM((CHUNKS, K), jnp.int32),
    ],
)
def _sc_degree(dst_hbm, out_hbm, deg_v, idx_v):
    c = lax.axis_index("c")
    s = lax.axis_index("s")
    wid = s * 2 + c
    pltpu.sync_copy(dst_hbm.at[wid], idx_v)

    def zero_body(i, carry):
        deg_v[pl.ds(i * 16, 16)] = jnp.zeros((16,), jnp.float32)
        return carry

    lax.fori_loop(0, NPAD // 16, zero_body, 0)

    ones16 = jnp.ones((16,), jnp.float32)

    def chunk_body(j, carry):
        for k in range(K // 16):
            iv = idx_v[j, pl.ds(k * 16, 16)]
            plsc.addupdate_scatter(deg_v, [iv], ones16)
        return carry

    lax.fori_loop(0, CHUNKS, chunk_body, 0)
    pltpu.sync_copy(deg_v, out_hbm.at[wid])


@functools.partial(
    pl.kernel,
    out_type=jax.ShapeDtypeStruct((2, NPAD, HID), jnp.float32),
    mesh=_sc_mesh,
    compiler_params=pltpu.CompilerParams(
        needs_layout_passes=False, use_tc_tiling_on_sc=False),
    scratch_types=[
        pltpu.VMEM_SHARED((NPAD, HID), jnp.float32),
        pltpu.VMEM((K, HID), jnp.float32),
        pltpu.VMEM((K, HID), jnp.float32),
        pltpu.VMEM((SEG, K), jnp.int32),
        pltpu.VMEM((SEG, K), jnp.int32),
        pltpu.SemaphoreType.DMA,
        pltpu.SemaphoreType.DMA,
    ],
)
def _sc_scatter(m_hbm, srca_hbm, srcb_hbm, dsta_hbm, dstb_hbm,
                out_hbm, acc, buf0, buf1, sidx, didx, sem0, sem1):
    c = lax.axis_index("c")
    s = lax.axis_index("s")
    r0 = s * RPS

    def zrow_body(i, carry):
        for f in range(HID // 16):
            buf0[i, pl.ds(f * 16, 16)] = jnp.zeros((16,), jnp.float32)
        return carry

    lax.fori_loop(0, K, zrow_body, 0)
    for z in range(RPS // K):
        pltpu.sync_copy(buf0, acc.at[pl.ds(r0 + z * K, K)])
    rem = RPS - (RPS // K) * K
    if rem:
        pltpu.sync_copy(buf0.at[pl.ds(0, rem)],
                        acc.at[pl.ds(r0 + (RPS // K) * K, rem)])
    plsc.subcore_barrier()

    def start(ch, buf, sem):
        pltpu.async_copy(m_hbm.at[sidx.at[ch]], buf, sem)

    def drain(buf, sem):
        pltpu.make_async_copy(m_hbm.at[pl.ds(0, K)], buf, sem).wait()

    def scat(ch, buf):
        pltpu.sync_copy(buf, acc.at[didx.at[ch]], add=True)

    def run(src_hbm, dst_hbm, nch):
        for g in range(nch // SEG):
            pltpu.sync_copy(src_hbm.at[s, pl.ds(g * SEG, SEG)], sidx)
            pltpu.sync_copy(dst_hbm.at[s, pl.ds(g * SEG, SEG)], didx)
            start(0, buf0, sem0)

            def body(i, carry):
                start(2 * i + 1, buf1, sem1)
                drain(buf0, sem0)
                scat(2 * i, buf0)
                start(2 * i + 2, buf0, sem0)
                drain(buf1, sem1)
                scat(2 * i + 1, buf1)
                return carry

            lax.fori_loop(0, SEG // 2 - 1, body, 0)
            start(SEG - 1, buf1, sem1)
            drain(buf0, sem0)
            scat(SEG - 2, buf0)
            drain(buf1, sem1)
            scat(SEG - 1, buf1)

    @pl.when(c == 0)
    def _run_a():
        run(srca_hbm, dsta_hbm, CH_A)

    @pl.when(c == 1)
    def _run_b():
        run(srcb_hbm, dstb_hbm, CH_B)

    plsc.subcore_barrier()
    pltpu.sync_copy(acc.at[pl.ds(r0, RPS)], out_hbm.at[c, pl.ds(r0, RPS)])


def _tc_prep(xp, degp, W1):
    def body(x_ref, dp_ref, w_ref, m1_ref, dinv_ref):
        deg = jnp.sum(dp_ref[...], axis=0, keepdims=True) + 1.0
        dinv = lax.rsqrt(deg)
        dinv_ref[...] = dinv
        dcol = jnp.reshape(dinv, (NPAD, 1))
        h0 = jnp.dot(x_ref[...], w_ref[...], preferred_element_type=jnp.float32)
        m1_ref[...] = h0 * dcol

    return pl.pallas_call(
        body,
        out_shape=[
            jax.ShapeDtypeStruct((NPAD, HID), jnp.float32),
            jax.ShapeDtypeStruct((1, NPAD), jnp.float32),
        ],
    )(xp, degp, W1)


def _tc_mid(s1p, m1, dinv, W2, b1):
    def body(s_ref, m_ref, d_ref, w_ref, b_ref, m2_ref):
        dcol = jnp.reshape(d_ref[...], (NPAD, 1))
        s = s_ref[0] + s_ref[1]
        h1 = jnp.maximum(dcol * (s + m_ref[...]) + b_ref[...], 0.0)
        m2 = jnp.dot(h1, w_ref[...], preferred_element_type=jnp.float32) * dcol
        rows = lax.broadcasted_iota(jnp.int32, (NPAD, 1), 0)
        m2_ref[...] = jnp.where(rows < NCAT, m2, 0.0)

    return pl.pallas_call(
        body,
        out_shape=jax.ShapeDtypeStruct((NPAD, HID), jnp.float32),
    )(s1p, m1, dinv, W2, b1)


def _tc_final(s2p, m2, dinv, b2, batchf, fc1_W, fc1_b, fc2_W, fc2_b):
    def body(s_ref, m_ref, d_ref, b2_ref, bat_ref, f1w_ref, f1b_ref,
             f2w_ref, f2b_ref, o_ref):
        dcol = jnp.reshape(d_ref[...], (NPAD, 1))
        s = s_ref[0] + s_ref[1]
        h2 = jnp.maximum(dcol * (s + m_ref[...]) + b2_ref[...], 0.0)
        gids = lax.broadcasted_iota(jnp.int32, (2 * G_PER, 1), 0).astype(jnp.float32)
        P = (gids == bat_ref[...]).astype(jnp.float32)
        sums = jnp.dot(P, h2, preferred_element_type=jnp.float32)
        cnt = jnp.sum(P, axis=1, keepdims=True)
        pooled = sums / jnp.maximum(cnt, 1.0)
        e1 = pooled[0:G_PER]
        e2 = pooled[G_PER:2 * G_PER]
        f1w = f1w_ref[...]
        z = (jnp.dot(e1, f1w[0:HID], preferred_element_type=jnp.float32)
             + jnp.dot(e2, f1w[HID:2 * HID], preferred_element_type=jnp.float32)
             + jnp.dot(jnp.abs(e1 - e2), f1w[2 * HID:3 * HID],
                       preferred_element_type=jnp.float32)
             + f1b_ref[...])
        z = jnp.maximum(z, 0.0)
        o = jnp.dot(z, f2w_ref[...], preferred_element_type=jnp.float32) + f2b_ref[...]
        o_ref[...] = jax.nn.sigmoid(o)

    return pl.pallas_call(
        body,
        out_shape=jax.ShapeDtypeStruct((G_PER, 1), jnp.float32),
    )(s2p, m2, dinv, b2, batchf, fc1_W, fc1_b, fc2_W, fc2_b)


def kernel(x1, edge_index1, batch1, x2, edge_index2, batch2,
           W1, b1, W2, b2, fc1_W, fc1_b, fc2_W, fc2_b):
    f32 = jnp.float32
    i32 = jnp.int32
    pad_e = EPAD - ECAT
    pad_n = NPAD - NCAT
    src = jnp.concatenate([edge_index1[0], edge_index2[0] + N_NODES,
                           jnp.full((pad_e,), NCAT, i32)])
    dst = jnp.concatenate([edge_index1[1], edge_index2[1] + N_NODES,
                           jnp.full((pad_e,), NCAT, i32)])
    ea = 16 * CH_A * K
    src_a = src[:ea].reshape(16, CH_A, K)
    src_b = src[ea:].reshape(16, CH_B, K)
    dst_a = dst[:ea].reshape(16, CH_A, K)
    dst_b = dst[ea:].reshape(16, CH_B, K)
    xp = jnp.concatenate([x1, x2, jnp.zeros((pad_n, F_IN), f32)], axis=0)
    batchf = jnp.concatenate(
        [batch1, batch2 + G_PER, jnp.full((pad_n,), 2 * G_PER + 7, i32)]
    ).astype(f32).reshape(1, NPAD)

    degp = _sc_degree(dst.reshape(NW, CHUNKS, K))
    m1, dinv = _tc_prep(xp, degp, W1)
    s1 = _sc_scatter(m1, src_a, src_b, dst_a, dst_b)
    m2 = _tc_mid(s1, m1, dinv, W2, b1.reshape(1, HID))
    s2 = _sc_scatter(m2, src_a, src_b, dst_a, dst_b)
    o = _tc_final(s2, m2, dinv, b2.reshape(1, HID), batchf,
                  fc1_W, fc1_b.reshape(1, HID), fc2_W, fc2_b.reshape(1, 1))
    return o[:, 0]
